# baseline (speedup 1.0000x reference)
import math

import jax
import jax.numpy as jnp
from jax.experimental import pallas as pl

N = 4096
H = 256
W = 256

_P = 1024
_NB = 512


def _raster_block(xyz_ref, scaling_ref, rotation_ref, feat_ref, out_ref):
    i = pl.program_id(0)
    n = pl.program_id(1)
    num_n = pl.num_programs(1)

    xc = 0.5 * (xyz_ref[0:1, :] + 1.0) * W
    yc = 0.5 * (xyz_ref[1:2, :] + 1.0) * H
    sx = jnp.abs(scaling_ref[0:1, :])
    sy = jnp.abs(scaling_ref[1:2, :])
    theta = jax.nn.sigmoid(rotation_ref[0:1, :]) * (2.0 * math.pi)
    c = jnp.cos(theta)
    sn = jnp.sin(theta)
    sx2 = sx * sx
    sy2 = sy * sy
    Sxx = c * c * sx2 + sn * sn * sy2
    Sxy = c * sn * (sx2 - sy2)
    Syy = sn * sn * sx2 + c * c * sy2
    det = Sxx * Syy - Sxy * Sxy
    inv = 1.0 / (det + 1e-12)
    cA = Syy * inv
    cB = -Sxy * inv
    cC = Sxx * inv

    p = jax.lax.broadcasted_iota(jnp.int32, (_P, 1), 0)
    px = (p & (W - 1)).astype(jnp.float32) + 0.5
    py = ((p >> 8) + i * (_P // W)).astype(jnp.float32) + 0.5

    dx = px - xc
    dy = py - yc
    sigma = 0.5 * (cA * dx * dx + cC * dy * dy) + cB * dx * dy
    vals = jnp.where(sigma >= 0.0, jnp.exp(-sigma), 0.0)

    contrib = jax.lax.dot_general(
        vals, feat_ref[...], (((1,), (1,)), ((), ())),
        preferred_element_type=jnp.float32)

    @pl.when(n == 0)
    def _():
        out_ref[...] = contrib

    @pl.when(n != 0)
    def _():
        out_ref[...] += contrib

    @pl.when(n == num_n - 1)
    def _():
        out_ref[...] = jnp.clip(out_ref[...], 0.0, 1.0)


def kernel(xyz, scaling, rotation, features, opacity):
    grid = (H * W // _P, N // _NB)
    img = pl.pallas_call(
        _raster_block,
        grid=grid,
        in_specs=[
            pl.BlockSpec((2, _NB), lambda i, n: (0, n)),
            pl.BlockSpec((2, _NB), lambda i, n: (0, n)),
            pl.BlockSpec((1, _NB), lambda i, n: (0, n)),
            pl.BlockSpec((3, _NB), lambda i, n: (0, n)),
        ],
        out_specs=pl.BlockSpec((_P, 3), lambda i, n: (i, 0)),
        out_shape=jax.ShapeDtypeStruct((H * W, 3), jnp.float32),
    )(xyz.T, scaling.T, rotation.T, (features * opacity).T)
    return img.reshape(1, H, W, 3).transpose(0, 3, 1, 2)
